# baseline (speedup 1.0000x reference)
import functools

import jax
import jax.numpy as jnp
from jax.experimental import pallas as pl
from jax.experimental.pallas import tpu as pltpu

N = 10000
IN_DIM = 128
H1 = 128
H2 = 64

ENC_R = 200
DEC_R = 200


def _enc_kernel(g_ref, f_ref, w1_ref, lns_ref, lnb_ref, w2_ref, b2_ref,
                w3_ref, b3_ref, eps_ref, mu_ref, logvar_ref, z_ref,
                support_ref):
    i = pl.program_id(0)

    @pl.when(i == 0)
    def _():
        support_ref[...] = jnp.dot(f_ref[...], w1_ref[...],
                                   preferred_element_type=jnp.float32)

    h1 = jnp.dot(g_ref[...], support_ref[...],
                 preferred_element_type=jnp.float32)
    h1 = jnp.maximum(h1, 0.0)
    mean = jnp.mean(h1, axis=-1, keepdims=True)
    var = jnp.mean((h1 - mean) ** 2, axis=-1, keepdims=True)
    h = (h1 - mean) / jnp.sqrt(var + 1e-5) * lns_ref[...] + lnb_ref[...]
    mu = jnp.dot(h, w2_ref[...], preferred_element_type=jnp.float32) + b2_ref[...]
    logvar = jnp.dot(h, w3_ref[...], preferred_element_type=jnp.float32) + b3_ref[...]
    z = eps_ref[...] * jnp.exp(logvar) + mu
    mu_ref[...] = mu
    logvar_ref[...] = logvar
    z_ref[...] = z


def _dec_kernel(zi_ref, zt_ref, adj_ref):
    adj_ref[...] = jnp.dot(zi_ref[...], zt_ref[...],
                           preferred_element_type=jnp.float32)


@jax.jit
def kernel(g, features, W1, ln_scale, ln_bias, W2, b2, W3, b3):
    eps = jax.random.normal(jax.random.key(42), (N, H2), dtype=jnp.float32)
    lns = ln_scale.reshape(1, H1)
    lnb = ln_bias.reshape(1, H1)
    b2r = b2.reshape(1, H2)
    b3r = b3.reshape(1, H2)

    n_blocks = N // ENC_R
    mu, logvar, z = pl.pallas_call(
        _enc_kernel,
        grid=(n_blocks,),
        in_specs=[
            pl.BlockSpec((ENC_R, N), lambda i: (i, 0)),
            pl.BlockSpec((N, IN_DIM), lambda i: (0, 0)),
            pl.BlockSpec((IN_DIM, H1), lambda i: (0, 0)),
            pl.BlockSpec((1, H1), lambda i: (0, 0)),
            pl.BlockSpec((1, H1), lambda i: (0, 0)),
            pl.BlockSpec((H1, H2), lambda i: (0, 0)),
            pl.BlockSpec((1, H2), lambda i: (0, 0)),
            pl.BlockSpec((H1, H2), lambda i: (0, 0)),
            pl.BlockSpec((1, H2), lambda i: (0, 0)),
            pl.BlockSpec((ENC_R, H2), lambda i: (i, 0)),
        ],
        out_specs=[
            pl.BlockSpec((ENC_R, H2), lambda i: (i, 0)),
            pl.BlockSpec((ENC_R, H2), lambda i: (i, 0)),
            pl.BlockSpec((ENC_R, H2), lambda i: (i, 0)),
        ],
        out_shape=[
            jax.ShapeDtypeStruct((N, H2), jnp.float32),
            jax.ShapeDtypeStruct((N, H2), jnp.float32),
            jax.ShapeDtypeStruct((N, H2), jnp.float32),
        ],
        scratch_shapes=[pltpu.VMEM((N, H1), jnp.float32)],
    )(g, features, W1, lns, lnb, W2, b2r, W3, b3r, eps)

    zt = z.T

    adj = pl.pallas_call(
        _dec_kernel,
        grid=(N // DEC_R,),
        in_specs=[
            pl.BlockSpec((DEC_R, H2), lambda i: (i, 0)),
            pl.BlockSpec((H2, N), lambda i: (0, 0)),
        ],
        out_specs=pl.BlockSpec((DEC_R, N), lambda i: (i, 0)),
        out_shape=jax.ShapeDtypeStruct((N, N), jnp.float32),
    )(z, zt)

    return (adj, mu, logvar, z)
